# Initial kernel scaffold; baseline (speedup 1.0000x reference)
import jax
import jax.numpy as jnp
from jax.experimental import pallas as pl


def kernel(x_gray_batch):
    raise NotImplementedError("write your pallas kernel here")


# speedup vs baseline: 8.3906x; 8.3906x over previous
import jax
import jax.numpy as jnp
from jax.experimental import pallas as pl
from jax.experimental.pallas import tpu as pltpu

L = 256
W = 512
ROWS_PER_CHUNK = 8
CHUNK = ROWS_PER_CHUNK * W
NC = W // ROWS_PER_CHUNK

_ANGLES = (
    (1, True, False),
    (513, True, False),
    (512, False, False),
    (511, False, True),
)


def _body(x_ref, xn_ref, o_ref, hist_ref):
    c = pl.program_id(1)

    @pl.when(c == 0)
    def _():
        hist_ref[...] = jnp.zeros_like(hist_ref)

    blk = jnp.concatenate([x_ref[0, 0], xn_ref[0, 0]], axis=0)
    a_val = blk[0:1, :]
    p = jax.lax.broadcasted_iota(jnp.int32, (1, CHUNK), 1)
    col = jnp.bitwise_and(p, W - 1)
    row = c * ROWS_PER_CHUNK + jax.lax.shift_right_logical(p, 9)
    row_ok = row < (W - 1)

    iota_lev = jax.lax.broadcasted_iota(jnp.int32, (L, CHUNK), 0)

    def onehot(v):
        return jnp.where(iota_lev == v, 1.0, 0.0).astype(jnp.bfloat16)

    b_hots = []
    for d, need_colhi, need_collo in _ANGLES:
        rolled = jnp.concatenate([blk[:, d:], blk[:, :d]], axis=1)
        bv = jnp.where(p < CHUNK - d, rolled[0:1, :], rolled[1:2, :])
        valid = row_ok if d != 1 else (col < W - 1)
        if need_colhi and d != 1:
            valid = valid & (col < W - 1)
        if need_collo:
            valid = valid & (col >= 1)
        b_hots.append(onehot(jnp.where(valid, bv, L + 7)))

    a_hot = onehot(a_val)
    b_all = jnp.concatenate(b_hots, axis=0)

    hist_ref[...] += jax.lax.dot_general(
        a_hot, b_all, (((1,), (1,)), ((), ())),
        preferred_element_type=jnp.float32)

    @pl.when(c == NC - 1)
    def _():
        fi = jax.lax.broadcasted_iota(jnp.int32, (L, L), 0).astype(jnp.float32)
        fj = jax.lax.broadcasted_iota(jnp.int32, (L, L), 1).astype(jnp.float32)
        dd = fi - fj
        d2 = dd * dd
        adist = jnp.abs(dd)
        hom_w = 1.0 / (1.0 + d2)
        eye = jnp.where(fi == fj, 1.0, 0.0)

        stats = [jnp.zeros((1, 1), jnp.float32) for _ in range(5)]
        for k in range(4):
            h = hist_ref[:, k * L:(k + 1) * L]
            t2 = 2.0 * jnp.sum(h, keepdims=True)
            contrast = 2.0 * jnp.sum(h * d2, keepdims=True) / t2
            dissim = 2.0 * jnp.sum(h * adist, keepdims=True) / t2
            homog = 2.0 * jnp.sum(h * hom_w, keepdims=True) / t2
            s1 = jnp.sum(h * h, keepdims=True)
            hh = jax.lax.dot_general(h, h, (((1,), (0,)), ((), ())),
                                     preferred_element_type=jnp.float32)
            s2 = jnp.sum(hh * eye, keepdims=True)
            energy = jnp.sqrt(2.0 * s1 + 2.0 * s2) / t2
            mu = jnp.sum(h * (fi + fj), keepdims=True) / t2
            di = fi - mu
            dj = fj - mu
            var = jnp.sum(h * (di * di + dj * dj), keepdims=True) / t2
            cov = 2.0 * jnp.sum(h * (di * dj), keepdims=True) / t2
            corr = jnp.where(var < 1e-15, 1.0, cov / jnp.maximum(var, 1e-30))
            for idx, v in enumerate((contrast, dissim, homog, energy, corr)):
                stats[idx] = stats[idx] + 0.25 * v

        zero = jnp.zeros((1, 1), jnp.float32)
        o_ref[...] = jnp.concatenate(
            stats + [zero, zero, zero], axis=1).reshape(1, 1, 8)


def kernel(x_gray_batch):
    b, h, w = x_gray_batch.shape
    xr = x_gray_batch.reshape(b, NC, 1, CHUNK)
    xr = jnp.pad(xr, ((0, 0), (0, 1), (0, 0), (0, 0)))
    out = pl.pallas_call(
        _body,
        grid=(b, NC),
        in_specs=[
            pl.BlockSpec((1, 1, 1, CHUNK), lambda bi, ci: (bi, ci, 0, 0)),
            pl.BlockSpec((1, 1, 1, CHUNK), lambda bi, ci: (bi, ci + 1, 0, 0)),
        ],
        out_specs=pl.BlockSpec((1, 1, 8), lambda bi, ci: (bi, 0, 0)),
        out_shape=jax.ShapeDtypeStruct((b, 1, 8), jnp.float32),
        scratch_shapes=[pltpu.VMEM((L, 4 * L), jnp.float32)],
        compiler_params=pltpu.CompilerParams(
            dimension_semantics=("parallel", "arbitrary"),
            vmem_limit_bytes=100 * 1024 * 1024,
        ),
    )(xr, xr)
    return out[:, 0, :5]


# speedup vs baseline: 8.9591x; 1.0678x over previous
import jax
import jax.numpy as jnp
from jax.experimental import pallas as pl
from jax.experimental.pallas import tpu as pltpu

L = 256
W = 512
ROWS_PER_CHUNK = 16
CHUNK = ROWS_PER_CHUNK * W
NC = W // ROWS_PER_CHUNK

_ANGLES = (
    (1, True, False),
    (513, True, False),
    (512, False, False),
    (511, False, True),
)


def _body(x_ref, xn_ref, o_ref, hist_ref):
    c = pl.program_id(1)

    @pl.when(c == 0)
    def _():
        hist_ref[...] = jnp.zeros_like(hist_ref)

    blk = jnp.concatenate([x_ref[0, 0], xn_ref[0, 0]], axis=0)
    a_val = blk[0:1, :]
    p = jax.lax.broadcasted_iota(jnp.int32, (1, CHUNK), 1)
    col = jnp.bitwise_and(p, W - 1)
    row = c * ROWS_PER_CHUNK + jax.lax.shift_right_logical(p, 9)
    row_ok = row < (W - 1)

    iota_lev = jax.lax.broadcasted_iota(jnp.int32, (L, CHUNK), 0)

    def onehot(v):
        return jnp.where(iota_lev == v, 1.0, 0.0).astype(jnp.bfloat16)

    b_hots = []
    for d, need_colhi, need_collo in _ANGLES:
        rolled = jnp.concatenate([blk[:, d:], blk[:, :d]], axis=1)
        bv = jnp.where(p < CHUNK - d, rolled[0:1, :], rolled[1:2, :])
        valid = row_ok if d != 1 else (col < W - 1)
        if need_colhi and d != 1:
            valid = valid & (col < W - 1)
        if need_collo:
            valid = valid & (col >= 1)
        b_hots.append(onehot(jnp.where(valid, bv, L + 7)))

    a_hot = onehot(a_val)
    b_all = jnp.concatenate(b_hots, axis=0)

    hist_ref[...] += jax.lax.dot_general(
        a_hot, b_all, (((1,), (1,)), ((), ())),
        preferred_element_type=jnp.float32)

    @pl.when(c == NC - 1)
    def _():
        fi = jax.lax.broadcasted_iota(jnp.int32, (L, L), 0).astype(jnp.float32)
        fj = jax.lax.broadcasted_iota(jnp.int32, (L, L), 1).astype(jnp.float32)
        dd = fi - fj
        d2 = dd * dd
        adist = jnp.abs(dd)
        hom_w = 1.0 / (1.0 + d2)
        eye = jnp.where(fi == fj, 1.0, 0.0)

        stats = [jnp.zeros((1, 1), jnp.float32) for _ in range(5)]
        for k in range(4):
            h = hist_ref[:, k * L:(k + 1) * L]
            t2 = 2.0 * jnp.sum(h, keepdims=True)
            contrast = 2.0 * jnp.sum(h * d2, keepdims=True) / t2
            dissim = 2.0 * jnp.sum(h * adist, keepdims=True) / t2
            homog = 2.0 * jnp.sum(h * hom_w, keepdims=True) / t2
            s1 = jnp.sum(h * h, keepdims=True)
            hh = jax.lax.dot_general(h, h, (((1,), (0,)), ((), ())),
                                     preferred_element_type=jnp.float32)
            s2 = jnp.sum(hh * eye, keepdims=True)
            energy = jnp.sqrt(2.0 * s1 + 2.0 * s2) / t2
            mu = jnp.sum(h * (fi + fj), keepdims=True) / t2
            di = fi - mu
            dj = fj - mu
            var = jnp.sum(h * (di * di + dj * dj), keepdims=True) / t2
            cov = 2.0 * jnp.sum(h * (di * dj), keepdims=True) / t2
            corr = jnp.where(var < 1e-15, 1.0, cov / jnp.maximum(var, 1e-30))
            for idx, v in enumerate((contrast, dissim, homog, energy, corr)):
                stats[idx] = stats[idx] + 0.25 * v

        zero = jnp.zeros((1, 1), jnp.float32)
        o_ref[...] = jnp.concatenate(
            stats + [zero, zero, zero], axis=1).reshape(1, 1, 8)


def kernel(x_gray_batch):
    b, h, w = x_gray_batch.shape
    xr = x_gray_batch.reshape(b, NC, 1, CHUNK)
    xr = jnp.pad(xr, ((0, 0), (0, 1), (0, 0), (0, 0)))
    out = pl.pallas_call(
        _body,
        grid=(b, NC),
        in_specs=[
            pl.BlockSpec((1, 1, 1, CHUNK), lambda bi, ci: (bi, ci, 0, 0)),
            pl.BlockSpec((1, 1, 1, CHUNK), lambda bi, ci: (bi, ci + 1, 0, 0)),
        ],
        out_specs=pl.BlockSpec((1, 1, 8), lambda bi, ci: (bi, 0, 0)),
        out_shape=jax.ShapeDtypeStruct((b, 1, 8), jnp.float32),
        scratch_shapes=[pltpu.VMEM((L, 4 * L), jnp.float32)],
        compiler_params=pltpu.CompilerParams(
            dimension_semantics=("parallel", "arbitrary"),
            vmem_limit_bytes=100 * 1024 * 1024,
        ),
    )(xr, xr)
    return out[:, 0, :5]


# speedup vs baseline: 9.2783x; 1.0356x over previous
import jax
import jax.numpy as jnp
from jax.experimental import pallas as pl
from jax.experimental.pallas import tpu as pltpu

L = 256
W = 512
ROWS_PER_CHUNK = 32
CHUNK = ROWS_PER_CHUNK * W
NC = W // ROWS_PER_CHUNK

_ANGLES = (
    (1, True, False),
    (513, True, False),
    (512, False, False),
    (511, False, True),
)


def _body(x_ref, xn_ref, o_ref, hist_ref):
    c = pl.program_id(1)

    @pl.when(c == 0)
    def _():
        hist_ref[...] = jnp.zeros_like(hist_ref)

    blk = jnp.concatenate([x_ref[0, 0], xn_ref[0, 0]], axis=0)
    a_val = blk[0:1, :]
    p = jax.lax.broadcasted_iota(jnp.int32, (1, CHUNK), 1)
    col = jnp.bitwise_and(p, W - 1)
    row = c * ROWS_PER_CHUNK + jax.lax.shift_right_logical(p, 9)
    row_ok = row < (W - 1)

    iota_lev = jax.lax.broadcasted_iota(jnp.int32, (L, CHUNK), 0)

    def onehot(v):
        return jnp.where(iota_lev == v, 1.0, 0.0).astype(jnp.bfloat16)

    b_hots = []
    for d, need_colhi, need_collo in _ANGLES:
        rolled = jnp.concatenate([blk[:, d:], blk[:, :d]], axis=1)
        bv = jnp.where(p < CHUNK - d, rolled[0:1, :], rolled[1:2, :])
        valid = row_ok if d != 1 else (col < W - 1)
        if need_colhi and d != 1:
            valid = valid & (col < W - 1)
        if need_collo:
            valid = valid & (col >= 1)
        b_hots.append(onehot(jnp.where(valid, bv, L + 7)))

    a_hot = onehot(a_val)
    b_all = jnp.concatenate(b_hots, axis=0)

    hist_ref[...] += jax.lax.dot_general(
        a_hot, b_all, (((1,), (1,)), ((), ())),
        preferred_element_type=jnp.float32)

    @pl.when(c == NC - 1)
    def _():
        fi = jax.lax.broadcasted_iota(jnp.int32, (L, L), 0).astype(jnp.float32)
        fj = jax.lax.broadcasted_iota(jnp.int32, (L, L), 1).astype(jnp.float32)
        dd = fi - fj
        d2 = dd * dd
        adist = jnp.abs(dd)
        hom_w = 1.0 / (1.0 + d2)
        eye = jnp.where(fi == fj, 1.0, 0.0)

        stats = [jnp.zeros((1, 1), jnp.float32) for _ in range(5)]
        for k in range(4):
            h = hist_ref[:, k * L:(k + 1) * L]
            t2 = 2.0 * jnp.sum(h, keepdims=True)
            contrast = 2.0 * jnp.sum(h * d2, keepdims=True) / t2
            dissim = 2.0 * jnp.sum(h * adist, keepdims=True) / t2
            homog = 2.0 * jnp.sum(h * hom_w, keepdims=True) / t2
            s1 = jnp.sum(h * h, keepdims=True)
            hh = jax.lax.dot_general(h, h, (((1,), (0,)), ((), ())),
                                     preferred_element_type=jnp.float32)
            s2 = jnp.sum(hh * eye, keepdims=True)
            energy = jnp.sqrt(2.0 * s1 + 2.0 * s2) / t2
            mu = jnp.sum(h * (fi + fj), keepdims=True) / t2
            di = fi - mu
            dj = fj - mu
            var = jnp.sum(h * (di * di + dj * dj), keepdims=True) / t2
            cov = 2.0 * jnp.sum(h * (di * dj), keepdims=True) / t2
            corr = jnp.where(var < 1e-15, 1.0, cov / jnp.maximum(var, 1e-30))
            for idx, v in enumerate((contrast, dissim, homog, energy, corr)):
                stats[idx] = stats[idx] + 0.25 * v

        zero = jnp.zeros((1, 1), jnp.float32)
        o_ref[...] = jnp.concatenate(
            stats + [zero, zero, zero], axis=1).reshape(1, 1, 8)


def kernel(x_gray_batch):
    b, h, w = x_gray_batch.shape
    xr = x_gray_batch.reshape(b, NC, 1, CHUNK)
    xr = jnp.pad(xr, ((0, 0), (0, 1), (0, 0), (0, 0)))
    out = pl.pallas_call(
        _body,
        grid=(b, NC),
        in_specs=[
            pl.BlockSpec((1, 1, 1, CHUNK), lambda bi, ci: (bi, ci, 0, 0)),
            pl.BlockSpec((1, 1, 1, CHUNK), lambda bi, ci: (bi, ci + 1, 0, 0)),
        ],
        out_specs=pl.BlockSpec((1, 1, 8), lambda bi, ci: (bi, 0, 0)),
        out_shape=jax.ShapeDtypeStruct((b, 1, 8), jnp.float32),
        scratch_shapes=[pltpu.VMEM((L, 4 * L), jnp.float32)],
        compiler_params=pltpu.CompilerParams(
            dimension_semantics=("parallel", "arbitrary"),
            vmem_limit_bytes=100 * 1024 * 1024,
        ),
    )(xr, xr)
    return out[:, 0, :5]
